# baseline (speedup 1.0000x reference)
import jax
import jax.numpy as jnp
from jax.experimental import pallas as pl
from jax.experimental.pallas import tpu as pltpu

_BLOCK = 32000


def _proj_body(x_ref, wv_ref, bv_ref, wo_ref, bo_ref, o_ref):
    o_ref[...] = x_ref[...] + bo_ref[...]


def kernel(node_features, rigids, edge_features, edge_index, Wg, bg, Wd, bd,
           Wqk, bqk, Wv, bv, Wo, bo):
    E, C = edge_features.shape
    block = _BLOCK if E % _BLOCK == 0 else E
    grid = (E // block,)
    bv2 = bv.reshape(1, C)
    bo2 = bo.reshape(1, C)
    return pl.pallas_call(
        _proj_body,
        grid=grid,
        in_specs=[
            pl.BlockSpec((block, C), lambda i: (i, 0)),
            pl.BlockSpec((C, C), lambda i: (0, 0)),
            pl.BlockSpec((1, C), lambda i: (0, 0)),
            pl.BlockSpec((C, C), lambda i: (0, 0)),
            pl.BlockSpec((1, C), lambda i: (0, 0)),
        ],
        out_specs=pl.BlockSpec((block, C), lambda i: (i, 0)),
        out_shape=jax.ShapeDtypeStruct((E, C), jnp.float32),
        compiler_params=pltpu.CompilerParams(
            dimension_semantics=("parallel",),
            vmem_limit_bytes=128 * 1024 * 1024,
        ),
    )(edge_features, Wv, bv2, Wo, bo2)


# speedup vs baseline: 1.0136x; 1.0136x over previous
import jax
import jax.numpy as jnp
from jax.experimental import pallas as pl
from jax.experimental.pallas import tpu as pltpu

_BLOCK = 32000


def _proj_body(x_ref, wv_ref, bv_ref, wo_ref, bo_ref, o_ref):
    wf = jnp.dot(wv_ref[...], wo_ref[...], preferred_element_type=jnp.float32)
    bf = jnp.dot(bv_ref[...], wo_ref[...], preferred_element_type=jnp.float32) + bo_ref[...]
    o_ref[...] = jnp.dot(x_ref[...], wf, preferred_element_type=jnp.float32) + bf


def kernel(node_features, rigids, edge_features, edge_index, Wg, bg, Wd, bd,
           Wqk, bqk, Wv, bv, Wo, bo):
    E, C = edge_features.shape
    block = _BLOCK if E % _BLOCK == 0 else E
    grid = (E // block,)
    bv2 = bv.reshape(1, C)
    bo2 = bo.reshape(1, C)
    return pl.pallas_call(
        _proj_body,
        grid=grid,
        in_specs=[
            pl.BlockSpec((block, C), lambda i: (i, 0)),
            pl.BlockSpec((C, C), lambda i: (0, 0)),
            pl.BlockSpec((1, C), lambda i: (0, 0)),
            pl.BlockSpec((C, C), lambda i: (0, 0)),
            pl.BlockSpec((1, C), lambda i: (0, 0)),
        ],
        out_specs=pl.BlockSpec((block, C), lambda i: (i, 0)),
        out_shape=jax.ShapeDtypeStruct((E, C), jnp.float32),
        compiler_params=pltpu.CompilerParams(
            dimension_semantics=("parallel",),
            vmem_limit_bytes=128 * 1024 * 1024,
        ),
    )(edge_features, Wv, bv2, Wo, bo2)
